# baseline (speedup 1.0000x reference)
import functools

import jax
import jax.numpy as jnp
from jax import lax
from jax.experimental import pallas as pl
from jax.experimental.pallas import tpu as pltpu
from jax.experimental.pallas import tpu_sc as plsc

N_NODES = 10000
N_EDGES = 320000
HID = 128

NC = 2
NS = 16
NW = NC * NS
EPW = N_EDGES // NW
CHUNK = 40
NCHUNK = EPW // CHUNK
NB = 3
N_PAD = 10240
ROWS_PER_TILE = N_PAD // NS


def _mlp3_body(n_in, *refs):
    x_refs = refs[:n_in]
    w1, b1, w2, b2, w3, b3, o_ref = refs[n_in:]
    u = x_refs[0][...]
    for r in x_refs[1:]:
        u = u + r[...]
    h = jnp.dot(u, w1[...], preferred_element_type=jnp.float32) + b1[...]
    h = jnp.maximum(h, 0.0)
    h = jnp.dot(h, w2[...], preferred_element_type=jnp.float32) + b2[...]
    h = jnp.maximum(h, 0.0)
    o_ref[...] = jnp.dot(h, w3[...], preferred_element_type=jnp.float32) + b3[...]


def _mlp3(xs, ps, block_rows):
    (w1, b1), (w2, b2), (w3, b3) = ps
    wb = (w1, b1.reshape(1, -1), w2, b2.reshape(1, -1), w3, b3.reshape(1, -1))
    rows = xs[0].shape[0]
    grid = (rows // block_rows,)
    in_specs = []
    for x in xs:
        d = x.shape[1]
        in_specs.append(pl.BlockSpec((block_rows, d), lambda i: (i, 0)))
    for w in wb:
        in_specs.append(pl.BlockSpec(w.shape, lambda i: (0, 0)))
    out_dim = w3.shape[1]
    return pl.pallas_call(
        functools.partial(_mlp3_body, len(xs)),
        grid=grid,
        in_specs=in_specs,
        out_specs=pl.BlockSpec((block_rows, out_dim), lambda i: (i, 0)),
        out_shape=jax.ShapeDtypeStruct((rows, out_dim), jnp.float32),
    )(*xs, *wb)


def _edge_body(h_hbm, ef_hbm, eidx_hbm, out_hbm,
               idxb, sidx, rows, efb, agg,
               i0, i1, i2, g0, g1, g2, e0, e1, e2, s0, s1, s2):
    cid = lax.axis_index("c")
    sid = lax.axis_index("s")
    wid = sid * NC + cid
    isem = (i0, i1, i2)
    gsem = (g0, g1, g2)
    esem = (e0, e1, e2)
    ssem = (s0, s1, s2)

    zeros16 = jnp.zeros((16,), jnp.float32)

    def zero_body(i, carry):
        r = i // 8
        v = i % 8
        rows[0, r, pl.ds(v * 16, 16)] = zeros16
        return carry

    lax.fori_loop(0, CHUNK * 8, zero_body, 0)
    for t in range(ROWS_PER_TILE // CHUNK):
        pltpu.sync_copy(rows.at[0],
                        agg.at[pl.ds(sid * ROWS_PER_TILE + t * CHUNK, CHUNK)])
    plsc.subcore_barrier()

    base = wid * EPW

    def start_idx(j, q):
        pltpu.async_copy(eidx_hbm.at[wid, j], idxb.at[q], isem[q])

    def wait_idx(q):
        pltpu.make_async_copy(eidx_hbm.at[0, 0], idxb.at[q], isem[q]).wait()

    def start_data(j, b):
        pltpu.async_copy(h_hbm.at[idxb.at[b, 0]], rows.at[b], gsem[b])
        pltpu.async_copy(ef_hbm.at[pl.ds(base + j * CHUNK, CHUNK)], efb.at[b],
                         esem[b])

    def finish(j, b):
        pltpu.make_async_copy(h_hbm.at[idxb.at[0, 0]], rows.at[b],
                              gsem[b]).wait()
        pltpu.make_async_copy(ef_hbm.at[pl.ds(0, CHUNK)], efb.at[b],
                              esem[b]).wait()

        @plsc.parallel_loop(0, CHUNK, step=1, unroll=2)
        def _(r):
            for v in range(8):
                sl = pl.ds(v * 16, 16)
                rows[b, r, sl] = jnp.maximum(rows[b, r, sl] + efb[b, r, sl], 0.0)
        sidx[b, pl.ds(0, 16)] = idxb[b, 1, pl.ds(0, 16)]
        sidx[b, pl.ds(16, 16)] = idxb[b, 1, pl.ds(16, 16)]
        sidx[b, pl.ds(24, 16)] = idxb[b, 1, pl.ds(24, 16)]
        pltpu.async_copy(rows.at[b], agg.at[sidx.at[b]], ssem[b], add=True)

    def wait_scatter(b):
        pltpu.make_async_copy(rows.at[b], agg.at[sidx.at[b]], ssem[b]).wait()

    for q in range(NB):
        start_idx(q, q)
    wait_idx(0)
    start_data(0, 0)
    wait_idx(1)
    start_data(1, 1)

    def pipe_body(s, carry):
        for u in range(NB):
            j = NB * s + u
            finish(j, u)

            @pl.when(j + NB < NCHUNK)
            def _():
                start_idx(j + NB, u)

            bn = (u + 2) % NB

            @pl.when(j + 2 < NCHUNK)
            def _():
                wait_idx(bn)

                @pl.when(j >= 1)
                def _():
                    wait_scatter(bn)

                start_data(j + 2, bn)

        return carry

    lax.fori_loop(0, NCHUNK // NB, pipe_body, 0)

    finish(NCHUNK - 1, (NCHUNK - 1) % NB)

    for b in range(NB):
        wait_scatter(b)

    plsc.subcore_barrier()
    pltpu.sync_copy(agg.at[pl.ds(sid * ROWS_PER_TILE, ROWS_PER_TILE)],
                    out_hbm.at[cid, pl.ds(sid * ROWS_PER_TILE, ROWS_PER_TILE)])


@functools.cache
def _edge_layer():
    return pl.kernel(
        _edge_body,
        out_type=jax.ShapeDtypeStruct((NC, N_PAD, HID), jnp.float32),
        mesh=plsc.VectorSubcoreMesh(core_axis_name="c", subcore_axis_name="s",
                                    num_cores=NC, num_subcores=NS),
        scratch_types=[
            pltpu.VMEM((NB, 2, CHUNK), jnp.int32),
            pltpu.VMEM((NB, CHUNK), jnp.int32),
            pltpu.VMEM((NB, CHUNK, HID), jnp.float32),
            pltpu.VMEM((NB, CHUNK, HID), jnp.float32),
            pltpu.VMEM_SHARED((N_PAD, HID), jnp.float32),
        ] + [pltpu.SemaphoreType.DMA] * 12,
    )


def kernel(x, edge_index, edge_attr, params):
    eidx = edge_index.reshape(2, NW, NCHUNK, CHUNK).transpose(1, 2, 0, 3)
    ef = _mlp3([edge_attr], params["edge_encoder"], block_rows=2000)
    h = _mlp3([x], params["node_proj"], block_rows=1000)
    for ps in params["gine"]:
        agg = _edge_layer()(h, ef, eidx)
        h = _mlp3([h, agg[0], agg[1]], ps, block_rows=1000)
    return _mlp3([h], params["out_proj"], block_rows=1000)


# speedup vs baseline: 1.1169x; 1.1169x over previous
import functools

import jax
import jax.numpy as jnp
from jax import lax
from jax.experimental import pallas as pl
from jax.experimental.pallas import tpu as pltpu
from jax.experimental.pallas import tpu_sc as plsc

N_NODES = 10000
N_EDGES = 320000
HID = 128

NC = 2
NS = 16
NW = NC * NS
EPW = N_EDGES // NW
CHUNK = 40
NCHUNK = EPW // CHUNK
NB = 3
N_PAD = 10240
ROWS_PER_TILE = N_PAD // NS


def _mlp3_body(with_agg, n_lin, *refs):
    if with_agg:
        x_ref, agg_ref = refs[:2]
        u = x_ref[...] + agg_ref[0] + agg_ref[1]
        rest = refs[2:]
    else:
        x_ref, = refs[:1]
        u = x_ref[...]
        rest = refs[1:]
    o_ref = rest[-1]
    h = u
    for k in range(n_lin):
        w, b = rest[2 * k], rest[2 * k + 1]
        h = jnp.dot(h, w[...], preferred_element_type=jnp.float32) + b[...]
        if k % 3 != 2:
            h = jnp.maximum(h, 0.0)
    o_ref[...] = h


def _mlp3(x, ps, block_rows, agg=None):
    wb = []
    for w, b in ps:
        wb.extend([w, b.reshape(1, -1)])
    wb = tuple(wb)
    rows = x.shape[0]
    grid = (rows // block_rows,)
    in_specs = [pl.BlockSpec((block_rows, x.shape[1]), lambda i: (i, 0))]
    args = [x]
    if agg is not None:
        in_specs.append(pl.BlockSpec((NC, block_rows, HID), lambda i: (0, i, 0)))
        args.append(agg)
    for w in wb:
        in_specs.append(pl.BlockSpec(w.shape, lambda i: (0, 0)))
    out_dim = wb[-2].shape[1]
    return pl.pallas_call(
        functools.partial(_mlp3_body, agg is not None, len(ps)),
        grid=grid,
        in_specs=in_specs,
        out_specs=pl.BlockSpec((block_rows, out_dim), lambda i: (i, 0)),
        out_shape=jax.ShapeDtypeStruct((rows, out_dim), jnp.float32),
    )(*args, *wb)


def _edge_body(h_hbm, ef_hbm, eidx_hbm, out_hbm,
               idxb, sidx, rows, efb, agg,
               i0, i1, i2, g0, g1, g2, e0, e1, e2, s0, s1, s2):
    cid = lax.axis_index("c")
    sid = lax.axis_index("s")
    wid = sid * NC + cid
    isem = (i0, i1, i2)
    gsem = (g0, g1, g2)
    esem = (e0, e1, e2)
    ssem = (s0, s1, s2)

    zeros16 = jnp.zeros((16,), jnp.float32)

    def zero_body(i, carry):
        r = i // 8
        v = i % 8
        rows[0, r, pl.ds(v * 16, 16)] = zeros16
        return carry

    lax.fori_loop(0, CHUNK * 8, zero_body, 0)
    for t in range(ROWS_PER_TILE // CHUNK):
        pltpu.sync_copy(rows.at[0],
                        agg.at[pl.ds(sid * ROWS_PER_TILE + t * CHUNK, CHUNK)])
    plsc.subcore_barrier()

    base = wid * EPW

    def start_idx(j, q):
        pltpu.async_copy(eidx_hbm.at[wid, j], idxb.at[q], isem[q])

    def wait_idx(q):
        pltpu.make_async_copy(eidx_hbm.at[0, 0], idxb.at[q], isem[q]).wait()

    def start_data(j, b):
        pltpu.async_copy(h_hbm.at[idxb.at[b, 0]], rows.at[b], gsem[b])
        pltpu.async_copy(ef_hbm.at[pl.ds(base + j * CHUNK, CHUNK)], efb.at[b],
                         esem[b])

    def finish(j, b):
        pltpu.make_async_copy(h_hbm.at[idxb.at[0, 0]], rows.at[b],
                              gsem[b]).wait()
        pltpu.make_async_copy(ef_hbm.at[pl.ds(0, CHUNK)], efb.at[b],
                              esem[b]).wait()

        def row_body(r, c2):
            for v in range(8):
                sl = pl.ds(v * 16, 16)
                rows[b, r, sl] = jnp.maximum(rows[b, r, sl] + efb[b, r, sl], 0.0)
            return c2

        lax.fori_loop(0, CHUNK, row_body, 0)
        sidx[b, pl.ds(0, 16)] = idxb[b, 1, pl.ds(0, 16)]
        sidx[b, pl.ds(16, 16)] = idxb[b, 1, pl.ds(16, 16)]
        sidx[b, pl.ds(24, 16)] = idxb[b, 1, pl.ds(24, 16)]
        pltpu.async_copy(rows.at[b], agg.at[sidx.at[b]], ssem[b], add=True)

    def wait_scatter(b):
        pltpu.make_async_copy(rows.at[b], agg.at[sidx.at[b]], ssem[b]).wait()

    for q in range(NB):
        start_idx(q, q)
    wait_idx(0)
    start_data(0, 0)
    wait_idx(1)
    start_data(1, 1)

    finish(0, 0)
    start_idx(NB, 0)
    wait_idx(2)
    start_data(2, 2)

    def pipe_body(s, carry):
        for u in range(NB):
            j = NB * s + u + 1
            b = (u + 1) % NB
            finish(j, b)
            start_idx(j + NB, b)
            bn = (u + 1 + 2) % NB
            wait_idx(bn)
            wait_scatter(bn)
            start_data(j + 2, bn)
        return carry

    lax.fori_loop(0, (NCHUNK - 4) // NB, pipe_body, 0)

    finish(247, 247 % NB)
    wait_idx(249 % NB)
    wait_scatter(249 % NB)
    start_data(249, 249 % NB)
    finish(248, 248 % NB)
    finish(249, 249 % NB)

    for b in range(NB):
        wait_scatter(b)

    plsc.subcore_barrier()
    pltpu.sync_copy(agg.at[pl.ds(sid * ROWS_PER_TILE, ROWS_PER_TILE)],
                    out_hbm.at[cid, pl.ds(sid * ROWS_PER_TILE, ROWS_PER_TILE)])


@functools.cache
def _edge_layer():
    return pl.kernel(
        _edge_body,
        out_type=jax.ShapeDtypeStruct((NC, N_PAD, HID), jnp.float32),
        mesh=plsc.VectorSubcoreMesh(core_axis_name="c", subcore_axis_name="s",
                                    num_cores=NC, num_subcores=NS),
        scratch_types=[
            pltpu.VMEM((NB, 2, CHUNK), jnp.int32),
            pltpu.VMEM((NB, CHUNK), jnp.int32),
            pltpu.VMEM((NB, CHUNK, HID), jnp.float32),
            pltpu.VMEM((NB, CHUNK, HID), jnp.float32),
            pltpu.VMEM_SHARED((N_PAD, HID), jnp.float32),
        ] + [pltpu.SemaphoreType.DMA] * 12,
    )


def kernel(x, edge_index, edge_attr, params):
    eidx = edge_index.reshape(2, NW, NCHUNK, CHUNK).transpose(1, 2, 0, 3)
    ef = _mlp3(edge_attr, params["edge_encoder"], block_rows=4000)
    h = _mlp3(x, params["node_proj"], block_rows=2000)
    for ps in params["gine"][:-1]:
        agg = _edge_layer()(h, ef, eidx)
        h = _mlp3(h, ps, block_rows=2000, agg=agg)
    agg = _edge_layer()(h, ef, eidx)
    return _mlp3(h, params["gine"][-1] + params["out_proj"],
                 block_rows=2000, agg=agg)
